# Initial kernel scaffold; baseline (speedup 1.0000x reference)
import jax
import jax.numpy as jnp
from jax.experimental import pallas as pl


def kernel(inputs, embed_tables, linear_w, lw_W, lw_b, W1, b1, W2, b2, Wf, bf):
    raise NotImplementedError("write your pallas kernel here")


# speedup vs baseline: 1.1866x; 1.1866x over previous
import functools

import jax
import jax.numpy as jnp
from jax import lax
from jax.experimental import pallas as pl
from jax.experimental.pallas import tpu as pltpu
from jax.experimental.pallas import tpu_sc as plsc

B = 16384
ND = 13
NS = 26
V = 100001
ED = 16
H1, H2 = 64, 32

NCORES = 2
NSUB = 16
NW = NCORES * NSUB
TOT = B * NS
PER_W = TOT // NW
CH = 1664
NCHUNK = PER_W // CH


def _sc_gather_build():
    mesh = plsc.VectorSubcoreMesh(core_axis_name="c", subcore_axis_name="s")

    @functools.partial(
        pl.kernel,
        mesh=mesh,
        compiler_params=pltpu.CompilerParams(use_tc_tiling_on_sc=False),
        out_type=(
            jax.ShapeDtypeStruct((TOT, ED), jnp.float32),
            jax.ShapeDtypeStruct((TOT,), jnp.float32),
        ),
        scratch_types=[
            pltpu.VMEM((CH,), jnp.int32),
            pltpu.VMEM((CH, ED), jnp.float32),
            pltpu.VMEM((CH,), jnp.float32),
            pltpu.SemaphoreType.DMA,
            pltpu.SemaphoreType.DMA,
        ],
    )
    def sc_gather(table_hbm, linw_hbm, idx_hbm, emb_out, wv_out,
                  idx_v, rows_v, wv_v, sem_r, sem_w):
        wid = lax.axis_index("s") * NCORES + lax.axis_index("c")
        base = wid * PER_W
        for c in range(NCHUNK):
            off = base + c * CH
            pltpu.sync_copy(idx_hbm.at[pl.ds(off, CH)], idx_v)
            cp_r = pltpu.async_copy(table_hbm.at[idx_v], rows_v, sem_r)
            cp_w = pltpu.async_copy(linw_hbm.at[idx_v], wv_v, sem_w)
            cp_r.wait()
            cp_w.wait()
            pltpu.sync_copy(rows_v, emb_out.at[pl.ds(off, CH)])
            pltpu.sync_copy(wv_v, wv_out.at[pl.ds(off, CH)])

    return sc_gather


_sc_gather = _sc_gather_build()

BT = 2048


def _tc_mlp_body(dense_ref, emb_ref, wv_ref, w1d_ref, w1e_ref, b1_ref,
                 w2_ref, b2_ref, wf_ref, bf_ref, lww_ref, lwb_ref, out_ref):
    dense = dense_ref[...]
    emb = emb_ref[...]
    wv = wv_ref[...]
    wide = (
        jnp.dot(dense, lww_ref[...], preferred_element_type=jnp.float32)
        + lwb_ref[...]
        + jnp.sum(wv, axis=1, keepdims=True)
    )
    h = jnp.dot(dense, w1d_ref[...], preferred_element_type=jnp.float32)
    h += jnp.dot(emb, w1e_ref[...], preferred_element_type=jnp.float32)
    h = jax.nn.relu(h + b1_ref[...])
    h = jax.nn.relu(
        jnp.dot(h, w2_ref[...], preferred_element_type=jnp.float32) + b2_ref[...]
    )
    deep = jnp.dot(h, wf_ref[...], preferred_element_type=jnp.float32) + bf_ref[...]
    out_ref[...] = jax.nn.sigmoid(0.5 * wide + 0.5 * deep)


def _tc_mlp(dense, emb, wv, w1d, w1e, b1, w2, b2, wf, bf, lww, lwb):
    rep = lambda shape: pl.BlockSpec(shape, lambda i: (0, 0))
    return pl.pallas_call(
        _tc_mlp_body,
        grid=(B // BT,),
        in_specs=[
            pl.BlockSpec((BT, ND), lambda i: (i, 0)),
            pl.BlockSpec((BT, NS * ED), lambda i: (i, 0)),
            pl.BlockSpec((BT, NS), lambda i: (i, 0)),
            rep((ND, H1)),
            rep((NS * ED, H1)),
            rep((1, H1)),
            rep((H1, H2)),
            rep((1, H2)),
            rep((H2, 1)),
            rep((1, 1)),
            rep((ND, 1)),
            rep((1, 1)),
        ],
        out_specs=pl.BlockSpec((BT, 1), lambda i: (i, 0)),
        out_shape=jax.ShapeDtypeStruct((B, 1), jnp.float32),
    )(dense, emb, wv, w1d, w1e, b1, w2, b2, wf, bf, lww, lwb)


def kernel(inputs, embed_tables, linear_w, lw_W, lw_b, W1, b1, W2, b2, Wf, bf):
    dense = inputs[:, :ND]
    sparse = inputs[:, ND:].astype(jnp.int32)
    offsets = (jnp.arange(NS, dtype=jnp.int32) * V)[None, :]
    idx = (sparse + offsets).reshape(-1)

    table = embed_tables.reshape(NS * V, ED)
    linw = linear_w.reshape(-1)
    emb_flat, wv_flat = _sc_gather(table, linw, idx)

    out = _tc_mlp(
        dense,
        emb_flat.reshape(B, NS * ED),
        wv_flat.reshape(B, NS),
        W1[:ND],
        W1[ND:],
        b1.reshape(1, H1),
        W2,
        b2.reshape(1, H2),
        Wf,
        bf.reshape(1, 1),
        lw_W,
        lw_b.reshape(1, 1),
    )
    return out


# speedup vs baseline: 3.2934x; 2.7754x over previous
import functools

import jax
import jax.numpy as jnp
from jax import lax
from jax.experimental import pallas as pl
from jax.experimental.pallas import tpu as pltpu
from jax.experimental.pallas import tpu_sc as plsc

B = 16384
ND = 13
NS = 26
V = 100001
VP = 100008
ED = 16
H1, H2 = 64, 32

NCORES = 2
NSUB = 16
NW = NCORES * NSUB
BCH = 1024
NCHUNK = B // BCH
NTASK = NS * NCHUNK
TPW = NTASK // NW


def _sc_gather_build():
    mesh = plsc.VectorSubcoreMesh(core_axis_name="c", subcore_axis_name="s")

    @functools.partial(
        pl.kernel,
        mesh=mesh,
        compiler_params=pltpu.CompilerParams(use_tc_tiling_on_sc=False),
        out_type=(
            jax.ShapeDtypeStruct((NS * ED, B), jnp.float32),
            jax.ShapeDtypeStruct((NS, B), jnp.float32),
        ),
        scratch_types=[
            pltpu.VMEM((BCH,), jnp.int32),
            pltpu.VMEM((BCH,), jnp.int32),
            pltpu.VMEM((ED, BCH), jnp.float32),
            pltpu.VMEM((BCH,), jnp.float32),
            pltpu.SemaphoreType.DMA,
            pltpu.SemaphoreType.DMA,
        ],
    )
    def sc_gather(table_hbm, linw_hbm, idx_hbm, idxw_hbm, embt_out, wv_out,
                  idx_v, idxw_v, rows_v, wv_v, sem_g, sem_w):
        wid = lax.axis_index("s") * NCORES + lax.axis_index("c")
        for t in range(TPW):
            task = wid * TPW + t
            s = task // NCHUNK
            boff = (task % NCHUNK) * BCH
            pltpu.sync_copy(idx_hbm.at[s, pl.ds(boff, BCH)], idx_v)
            pltpu.sync_copy(idxw_hbm.at[s, pl.ds(boff, BCH)], idxw_v)
            cps = [
                pltpu.async_copy(table_hbm.at[s * ED + e].at[idx_v],
                                 rows_v.at[e], sem_g)
                for e in range(ED)
            ]
            cpw = pltpu.async_copy(linw_hbm.at[idxw_v], wv_v, sem_w)
            for cp in cps:
                cp.wait()
            cpw.wait()
            for e in range(ED):
                pltpu.sync_copy(rows_v.at[e],
                                embt_out.at[s * ED + e, pl.ds(boff, BCH)])
            pltpu.sync_copy(wv_v, wv_out.at[s, pl.ds(boff, BCH)])

    return sc_gather


_sc_gather = _sc_gather_build()

BT = 2048


def _tc_mlp_body(xt_ref, embt_ref, wv_ref, w1dt_ref, w1et_ref, b1_ref,
                 w2t_ref, b2_ref, wft_ref, bf_ref, lwwt_ref, lwb_ref, out_ref):
    dense_t = xt_ref[:ND, :]
    emb_t = embt_ref[...]
    wv_t = wv_ref[...]
    wide = (
        jnp.dot(lwwt_ref[...], dense_t, preferred_element_type=jnp.float32)
        + lwb_ref[...]
        + jnp.sum(wv_t, axis=0, keepdims=True)
    )
    h = jnp.dot(w1dt_ref[...], dense_t, preferred_element_type=jnp.float32)
    h += jnp.dot(w1et_ref[...], emb_t, preferred_element_type=jnp.float32)
    h = jax.nn.relu(h + b1_ref[...])
    h = jax.nn.relu(
        jnp.dot(w2t_ref[...], h, preferred_element_type=jnp.float32) + b2_ref[...]
    )
    deep = jnp.dot(wft_ref[...], h, preferred_element_type=jnp.float32) + bf_ref[...]
    out_ref[...] = jax.nn.sigmoid(0.5 * wide + 0.5 * deep)


def _tc_mlp(xt, embt, wvt, w1dt, w1et, b1c, w2t, b2c, wft, bfc, lwwt, lwbc):
    rep = lambda shape: pl.BlockSpec(shape, lambda i: (0, 0))
    return pl.pallas_call(
        _tc_mlp_body,
        grid=(B // BT,),
        in_specs=[
            pl.BlockSpec((ND + NS, BT), lambda i: (0, i)),
            pl.BlockSpec((NS * ED, BT), lambda i: (0, i)),
            pl.BlockSpec((NS, BT), lambda i: (0, i)),
            rep((H1, ND)),
            rep((H1, NS * ED)),
            rep((H1, 1)),
            rep((H2, H1)),
            rep((H2, 1)),
            rep((1, H2)),
            rep((1, 1)),
            rep((1, ND)),
            rep((1, 1)),
        ],
        out_specs=pl.BlockSpec((1, BT), lambda i: (0, i)),
        out_shape=jax.ShapeDtypeStruct((1, B), jnp.float32),
    )(xt, embt, wvt, w1dt, w1et, b1c, w2t, b2c, wft, bfc, lwwt, lwbc)


def kernel(inputs, embed_tables, linear_w, lw_W, lw_b, W1, b1, W2, b2, Wf, bf):
    table_t = jnp.transpose(embed_tables, (0, 2, 1))
    table_f = jnp.pad(table_t, ((0, 0), (0, 0), (0, VP - V))).reshape(NS * ED, VP)
    linw_f = linear_w.reshape(-1)

    xt = jnp.transpose(inputs, (1, 0))
    idx_t = xt[ND:, :].astype(jnp.int32)
    idxw_t = idx_t + (jnp.arange(NS, dtype=jnp.int32) * V)[:, None]

    embt, wvt = _sc_gather(table_f, linw_f, idx_t, idxw_t)

    out_row = _tc_mlp(
        xt,
        embt,
        wvt,
        W1[:ND].T,
        W1[ND:].T,
        b1.reshape(H1, 1),
        W2.T,
        b2.reshape(H2, 1),
        Wf.T,
        bf.reshape(1, 1),
        lw_W.T,
        lw_b.reshape(1, 1),
    )
    return out_row.reshape(B, 1)


# speedup vs baseline: 13.7192x; 4.1657x over previous
import functools

import jax
import jax.numpy as jnp
from jax import lax
from jax.experimental import pallas as pl
from jax.experimental.pallas import tpu as pltpu
from jax.experimental.pallas import tpu_sc as plsc

B = 16384
ND = 13
NS = 26
V = 100001
VP = 100096
ED = 16
H1, H2 = 64, 32

NCORES = 2
NSUB = 16
NW = NCORES * NSUB
BCH = 1024
NCHUNK = B // BCH
NTASK = NS * NCHUNK
TPW = NTASK // NW

ROWS_PER_FIELD = ED * (VP // 128)
SCRATCH_ROWS = NS * ROWS_PER_FIELD


def _detile_body(in_ref, out_ref):
    out_ref[...] = in_ref[0].reshape(ROWS_PER_FIELD, 128)


def _tc_detile(table_t):
    return pl.pallas_call(
        _detile_body,
        grid=(NS,),
        in_specs=[pl.BlockSpec((1, ED, VP), lambda s: (s, 0, 0))],
        out_specs=pl.BlockSpec((ROWS_PER_FIELD, 128), lambda s: (s, 0)),
        out_shape=jax.ShapeDtypeStruct((SCRATCH_ROWS, 128), jnp.float32),
    )(table_t)


def _sc_gather_build():
    mesh = plsc.VectorSubcoreMesh(core_axis_name="c", subcore_axis_name="s")

    @functools.partial(
        pl.kernel,
        mesh=mesh,
        compiler_params=pltpu.CompilerParams(use_tc_tiling_on_sc=False),
        out_type=(
            jax.ShapeDtypeStruct((NS * ED, B), jnp.float32),
            jax.ShapeDtypeStruct((NS, B), jnp.float32),
        ),

        scratch_types=[
            pltpu.VMEM((BCH,), jnp.int32),
            pltpu.VMEM((BCH,), jnp.int32),
            pltpu.VMEM((ED, BCH), jnp.float32),
            pltpu.VMEM((BCH,), jnp.float32),
            pltpu.SemaphoreType.DMA,
            pltpu.SemaphoreType.DMA,
        ],
    )
    def sc_gather(table_hbm, linw_hbm, idx_hbm, idxw_hbm, embt_out, wv_out,
                  idx_v, idxw_v, rows_v, wv_v, sem_g, sem_w):
        wid = lax.axis_index("s") * NCORES + lax.axis_index("c")
        for t in range(TPW):
            task = wid * TPW + t
            s = task // NCHUNK
            boff = (task % NCHUNK) * BCH
            pltpu.sync_copy(idx_hbm.at[s, pl.ds(boff, BCH)], idx_v)
            pltpu.sync_copy(idxw_hbm.at[s, pl.ds(boff, BCH)], idxw_v)
            cps = [
                pltpu.async_copy(
                    table_hbm.at[pl.ds((s * ED + e) * VP, VP)].at[idx_v],
                    rows_v.at[e], sem_g)
                for e in range(ED)
            ]
            cpw = pltpu.async_copy(linw_hbm.at[0].at[idxw_v], wv_v, sem_w)
            for cp in cps:
                cp.wait()
            cpw.wait()
            for e in range(ED):
                pltpu.sync_copy(rows_v.at[e],
                                embt_out.at[s * ED + e, pl.ds(boff, BCH)])
            pltpu.sync_copy(wv_v, wv_out.at[s, pl.ds(boff, BCH)])

    return sc_gather


_sc_gather = _sc_gather_build()

BT = 2048


def _tc_mlp_body(xt_ref, embt_ref, wv_ref, w1dt_ref, w1et_ref, b1_ref,
                 w2t_ref, b2_ref, wft_ref, bf_ref, lwwt_ref, lwb_ref, out_ref):
    dense_t = xt_ref[:ND, :]
    emb_t = embt_ref[...]
    wv_t = wv_ref[...]
    wide = (
        jnp.dot(lwwt_ref[...], dense_t, preferred_element_type=jnp.float32)
        + lwb_ref[...]
        + jnp.sum(wv_t, axis=0, keepdims=True)
    )
    h = jnp.dot(w1dt_ref[...], dense_t, preferred_element_type=jnp.float32)
    h += jnp.dot(w1et_ref[...], emb_t, preferred_element_type=jnp.float32)
    h = jax.nn.relu(h + b1_ref[...])
    h = jax.nn.relu(
        jnp.dot(w2t_ref[...], h, preferred_element_type=jnp.float32) + b2_ref[...]
    )
    deep = jnp.dot(wft_ref[...], h, preferred_element_type=jnp.float32) + bf_ref[...]
    out_ref[...] = jax.nn.sigmoid(0.5 * wide + 0.5 * deep)


def _tc_mlp(xt, embt, wvt, w1dt, w1et, b1c, w2t, b2c, wft, bfc, lwwt, lwbc):
    rep = lambda shape: pl.BlockSpec(shape, lambda i: (0, 0))
    return pl.pallas_call(
        _tc_mlp_body,
        grid=(B // BT,),
        in_specs=[
            pl.BlockSpec((ND + NS, BT), lambda i: (0, i)),
            pl.BlockSpec((NS * ED, BT), lambda i: (0, i)),
            pl.BlockSpec((NS, BT), lambda i: (0, i)),
            rep((H1, ND)),
            rep((H1, NS * ED)),
            rep((H1, 1)),
            rep((H2, H1)),
            rep((H2, 1)),
            rep((1, H2)),
            rep((1, 1)),
            rep((1, ND)),
            rep((1, 1)),
        ],
        out_specs=pl.BlockSpec((1, BT), lambda i: (0, i)),
        out_shape=jax.ShapeDtypeStruct((1, B), jnp.float32),
    )(xt, embt, wvt, w1dt, w1et, b1c, w2t, b2c, wft, bfc, lwwt, lwbc)


def kernel(inputs, embed_tables, linear_w, lw_W, lw_b, W1, b1, W2, b2, Wf, bf):
    table_t = jnp.transpose(embed_tables, (0, 2, 1))
    table_f = _tc_detile(table_t).reshape(-1)
    linw_t = jnp.transpose(linear_w, (1, 0))

    xt = jnp.transpose(inputs, (1, 0))
    idx_t = xt[ND:, :].astype(jnp.int32)
    idxw_t = idx_t + (jnp.arange(NS, dtype=jnp.int32) * V)[:, None]

    embt, wvt = _sc_gather(table_f, linw_t, idx_t, idxw_t)

    out_row = _tc_mlp(
        xt,
        embt,
        wvt,
        W1[:ND].T,
        W1[ND:].T,
        b1.reshape(H1, 1),
        W2.T,
        b2.reshape(H2, 1),
        Wf.T,
        bf.reshape(1, 1),
        lw_W.T,
        lw_b.reshape(1, 1),
    )
    return out_row.reshape(B, 1)


# speedup vs baseline: 13.7892x; 1.0051x over previous
import functools

import jax
import jax.numpy as jnp
from jax import lax
from jax.experimental import pallas as pl
from jax.experimental.pallas import tpu as pltpu
from jax.experimental.pallas import tpu_sc as plsc

B = 16384
ND = 13
NS = 26
V = 100001
VP = 100096
ED = 16
H1, H2 = 64, 32

NCORES = 2
NSUB = 16
NW = NCORES * NSUB

NG = 2
NF = NS // NG
BCH = 1024
NCHUNK = B // BCH
NTASK = NF * NCHUNK
TPW = -(-NTASK // NW)

ROWS_PER_FIELD = ED * (VP // 128)
SCRATCH_ROWS = NF * ROWS_PER_FIELD


def _detile_body(in_ref, out_ref):
    out_ref[...] = in_ref[0].reshape(ROWS_PER_FIELD, 128)


def _tc_detile(table_t, lo):
    return pl.pallas_call(
        _detile_body,
        grid=(NF,),
        in_specs=[pl.BlockSpec((1, ED, VP), lambda s: (s + lo, 0, 0))],
        out_specs=pl.BlockSpec((ROWS_PER_FIELD, 128), lambda s: (s, 0)),
        out_shape=jax.ShapeDtypeStruct((SCRATCH_ROWS, 128), jnp.float32),
    )(table_t)


def _sc_gather_build():
    mesh = plsc.VectorSubcoreMesh(core_axis_name="c", subcore_axis_name="s")

    @functools.partial(
        pl.kernel,
        mesh=mesh,
        compiler_params=pltpu.CompilerParams(use_tc_tiling_on_sc=False),
        out_type=(
            jax.ShapeDtypeStruct((NF * ED, B), jnp.float32),
            jax.ShapeDtypeStruct((NF, B), jnp.float32),
        ),
        scratch_types=[
            pltpu.VMEM((BCH,), jnp.int32),
            pltpu.VMEM((BCH,), jnp.int32),
            pltpu.VMEM((ED, BCH), jnp.float32),
            pltpu.VMEM((BCH,), jnp.float32),
            pltpu.SemaphoreType.DMA,
            pltpu.SemaphoreType.DMA,
        ],
    )
    def sc_gather(table_hbm, linw_hbm, idx_hbm, idxw_hbm, embt_out, wv_out,
                  idx_v, idxw_v, rows_v, wv_v, sem_g, sem_w):
        wid = lax.axis_index("s") * NCORES + lax.axis_index("c")
        for t in range(TPW):
            task = t * NW + wid

            @pl.when(task < NTASK)
            def _():
                s = task // NCHUNK
                boff = (task % NCHUNK) * BCH
                pltpu.sync_copy(idx_hbm.at[s, pl.ds(boff, BCH)], idx_v)
                pltpu.sync_copy(idxw_hbm.at[s, pl.ds(boff, BCH)], idxw_v)
                cps = [
                    pltpu.async_copy(
                        table_hbm.at[pl.ds((s * ED + e) * VP, VP)].at[idx_v],
                        rows_v.at[e], sem_g)
                    for e in range(ED)
                ]
                cpw = pltpu.async_copy(linw_hbm.at[0].at[idxw_v], wv_v, sem_w)
                for cp in cps:
                    cp.wait()
                cpw.wait()
                for e in range(ED):
                    pltpu.sync_copy(rows_v.at[e],
                                    embt_out.at[s * ED + e, pl.ds(boff, BCH)])
                pltpu.sync_copy(wv_v, wv_out.at[s, pl.ds(boff, BCH)])

    return sc_gather


_sc_gather = _sc_gather_build()

BT = 2048


def _tc_mlp_body(xt_ref, emba_ref, embb_ref, wva_ref, wvb_ref,
                 w1dt_ref, w1at_ref, w1bt_ref, b1_ref,
                 w2t_ref, b2_ref, wft_ref, bf_ref, lwwt_ref, lwb_ref, out_ref):
    dense_t = xt_ref[:ND, :]
    wide = (
        jnp.dot(lwwt_ref[...], dense_t, preferred_element_type=jnp.float32)
        + lwb_ref[...]
        + jnp.sum(wva_ref[...], axis=0, keepdims=True)
        + jnp.sum(wvb_ref[...], axis=0, keepdims=True)
    )
    h = jnp.dot(w1dt_ref[...], dense_t, preferred_element_type=jnp.float32)
    h += jnp.dot(w1at_ref[...], emba_ref[...], preferred_element_type=jnp.float32)
    h += jnp.dot(w1bt_ref[...], embb_ref[...], preferred_element_type=jnp.float32)
    h = jax.nn.relu(h + b1_ref[...])
    h = jax.nn.relu(
        jnp.dot(w2t_ref[...], h, preferred_element_type=jnp.float32) + b2_ref[...]
    )
    deep = jnp.dot(wft_ref[...], h, preferred_element_type=jnp.float32) + bf_ref[...]
    out_ref[...] = jax.nn.sigmoid(0.5 * wide + 0.5 * deep)


def _tc_mlp(xt, emba, embb, wva, wvb, w1dt, w1at, w1bt, b1c, w2t, b2c,
            wft, bfc, lwwt, lwbc):
    rep = lambda shape: pl.BlockSpec(shape, lambda i: (0, 0))
    return pl.pallas_call(
        _tc_mlp_body,
        grid=(B // BT,),
        in_specs=[
            pl.BlockSpec((ND + NS, BT), lambda i: (0, i)),
            pl.BlockSpec((NF * ED, BT), lambda i: (0, i)),
            pl.BlockSpec((NF * ED, BT), lambda i: (0, i)),
            pl.BlockSpec((NF, BT), lambda i: (0, i)),
            pl.BlockSpec((NF, BT), lambda i: (0, i)),
            rep((H1, ND)),
            rep((H1, NF * ED)),
            rep((H1, NF * ED)),
            rep((H1, 1)),
            rep((H2, H1)),
            rep((H2, 1)),
            rep((1, H2)),
            rep((1, 1)),
            rep((1, ND)),
            rep((1, 1)),
        ],
        out_specs=pl.BlockSpec((1, BT), lambda i: (0, i)),
        out_shape=jax.ShapeDtypeStruct((1, B), jnp.float32),
    )(xt, emba, embb, wva, wvb, w1dt, w1at, w1bt, b1c, w2t, b2c,
      wft, bfc, lwwt, lwbc)


def kernel(inputs, embed_tables, linear_w, lw_W, lw_b, W1, b1, W2, b2, Wf, bf):
    table_t = jnp.transpose(embed_tables, (0, 2, 1))
    linw_t = jnp.transpose(linear_w, (1, 0))

    xt = jnp.transpose(inputs, (1, 0))
    idx_t = xt[ND:, :].astype(jnp.int32)
    idxw_t = idx_t + (jnp.arange(NS, dtype=jnp.int32) * V)[:, None]

    halves = []
    for g in range(NG):
        lo = g * NF
        table_f = _tc_detile(table_t, lo).reshape(-1)
        halves.append(
            _sc_gather(table_f, linw_t,
                       idx_t[lo:lo + NF], idxw_t[lo:lo + NF])
        )
    (emba, wva), (embb, wvb) = halves

    out_row = _tc_mlp(
        xt,
        emba,
        embb,
        wva,
        wvb,
        W1[:ND].T,
        W1[ND:ND + NF * ED].T,
        W1[ND + NF * ED:].T,
        b1.reshape(H1, 1),
        W2.T,
        b2.reshape(H2, 1),
        Wf.T,
        bf.reshape(1, 1),
        lw_W.T,
        lw_b.reshape(1, 1),
    )
    return out_row.reshape(B, 1)


# speedup vs baseline: 14.1192x; 1.0239x over previous
import functools

import jax
import jax.numpy as jnp
from jax import lax
from jax.experimental import pallas as pl
from jax.experimental.pallas import tpu as pltpu
from jax.experimental.pallas import tpu_sc as plsc

B = 16384
ND = 13
NS = 26
V = 100001
VP = 100096
ED = 16
H1, H2 = 64, 32

NCORES = 2
NSUB = 16
NW = NCORES * NSUB

NG = 2
NF = NS // NG
BCH = 1024
NCHUNK = B // BCH
NTASK = NF * NCHUNK
TPW = -(-NTASK // NW)

ROWS_PER_FIELD = ED * (VP // 128)
SCRATCH_ROWS = NF * ROWS_PER_FIELD


def _detile_body(in_ref, out_ref):
    out_ref[...] = in_ref[0].reshape(ROWS_PER_FIELD, 128)


def _tc_detile(table_t, lo):
    return pl.pallas_call(
        _detile_body,
        grid=(NF,),
        in_specs=[pl.BlockSpec((1, ED, VP), lambda s: (s + lo, 0, 0))],
        out_specs=pl.BlockSpec((ROWS_PER_FIELD, 128), lambda s: (s, 0)),
        out_shape=jax.ShapeDtypeStruct((SCRATCH_ROWS, 128), jnp.float32),
    )(table_t)


def _sc_gather_build():
    mesh = plsc.VectorSubcoreMesh(core_axis_name="c", subcore_axis_name="s")

    @functools.partial(
        pl.kernel,
        mesh=mesh,
        compiler_params=pltpu.CompilerParams(use_tc_tiling_on_sc=False),
        out_type=(
            jax.ShapeDtypeStruct((NF * ED, B), jnp.float32),
            jax.ShapeDtypeStruct((NF, B), jnp.float32),
        ),
        scratch_types=[
            pltpu.VMEM((BCH,), jnp.int32),
            pltpu.VMEM((BCH,), jnp.int32),
            pltpu.VMEM((ED, BCH), jnp.float32),
            pltpu.VMEM((BCH,), jnp.float32),
            pltpu.SemaphoreType.DMA,
            pltpu.SemaphoreType.DMA,
        ],
    )
    def sc_gather(table_hbm, linw_hbm, idx_hbm, idxw_hbm, embt_out, wv_out,
                  idx_v, idxw_v, rows_v, wv_v, sem_g, sem_w):
        wid = lax.axis_index("s") * NCORES + lax.axis_index("c")
        for t in range(TPW):
            task = t * NW + wid

            @pl.when(task < NTASK)
            def _():
                s = task // NCHUNK
                boff = (task % NCHUNK) * BCH
                pltpu.sync_copy(idx_hbm.at[s, pl.ds(boff, BCH)], idx_v)
                pltpu.sync_copy(idxw_hbm.at[s, pl.ds(boff, BCH)], idxw_v)
                cps = [
                    pltpu.async_copy(
                        table_hbm.at[pl.ds((s * ED + e) * VP, VP)].at[idx_v],
                        rows_v.at[e], sem_g)
                    for e in range(ED)
                ]
                cpw = pltpu.async_copy(linw_hbm.at[0].at[idxw_v], wv_v, sem_w)
                for cp in cps:
                    cp.wait()
                cpw.wait()
                for e in range(ED):
                    pltpu.sync_copy(rows_v.at[e],
                                    embt_out.at[s * ED + e, pl.ds(boff, BCH)])
                pltpu.sync_copy(wv_v, wv_out.at[s, pl.ds(boff, BCH)])

    return sc_gather


_sc_gather = _sc_gather_build()

def _tc_mlp_body(xt_ref, emba_ref, embb_ref, wva_ref, wvb_ref,
                 w1dt_ref, w1at_ref, w1bt_ref, b1_ref,
                 w2t_ref, b2_ref, wft_ref, bf_ref, lwwt_ref, lwb_ref, out_ref):
    dense_t = xt_ref[:ND, :]
    emba = emba_ref[...].reshape(NF * ED, 128, 128).reshape(NF * ED, B)
    embb = embb_ref[...].reshape(NF * ED, 128, 128).reshape(NF * ED, B)
    wva = wva_ref[...].reshape(NF, 128, 128).reshape(NF, B)
    wvb = wvb_ref[...].reshape(NF, 128, 128).reshape(NF, B)
    wide = (
        jnp.dot(lwwt_ref[...], dense_t, preferred_element_type=jnp.float32)
        + lwb_ref[...]
        + jnp.sum(wva, axis=0, keepdims=True)
        + jnp.sum(wvb, axis=0, keepdims=True)
    )
    h = jnp.dot(w1dt_ref[...], dense_t, preferred_element_type=jnp.float32)
    h += jnp.dot(w1at_ref[...], emba, preferred_element_type=jnp.float32)
    h += jnp.dot(w1bt_ref[...], embb, preferred_element_type=jnp.float32)
    h = jax.nn.relu(h + b1_ref[...])
    h = jax.nn.relu(
        jnp.dot(w2t_ref[...], h, preferred_element_type=jnp.float32) + b2_ref[...]
    )
    deep = jnp.dot(wft_ref[...], h, preferred_element_type=jnp.float32) + bf_ref[...]
    out_ref[...] = jax.nn.sigmoid(0.5 * wide + 0.5 * deep)


def _tc_mlp(xt, emba, embb, wva, wvb, w1dt, w1at, w1bt, b1c, w2t, b2c,
            wft, bfc, lwwt, lwbc):
    return pl.pallas_call(
        _tc_mlp_body,
        out_shape=jax.ShapeDtypeStruct((1, B), jnp.float32),
    )(xt, emba.reshape(NF * ED * 128, 128), embb.reshape(NF * ED * 128, 128),
      wva.reshape(NF * 128, 128), wvb.reshape(NF * 128, 128),
      w1dt, w1at, w1bt, b1c, w2t, b2c, wft, bfc, lwwt, lwbc)


def kernel(inputs, embed_tables, linear_w, lw_W, lw_b, W1, b1, W2, b2, Wf, bf):
    table_t = jnp.transpose(embed_tables, (0, 2, 1))
    linw_t = jnp.transpose(linear_w, (1, 0))

    xt = jnp.transpose(inputs, (1, 0))
    idx_t = xt[ND:, :].astype(jnp.int32)
    idxw_t = idx_t + (jnp.arange(NS, dtype=jnp.int32) * V)[:, None]

    halves = []
    for g in range(NG):
        lo = g * NF
        table_f = _tc_detile(table_t, lo).reshape(-1)
        halves.append(
            _sc_gather(table_f, linw_t,
                       idx_t[lo:lo + NF], idxw_t[lo:lo + NF])
        )
    (emba, wva), (embb, wvb) = halves

    out_row = _tc_mlp(
        xt,
        emba,
        embb,
        wva,
        wvb,
        W1[:ND].T,
        W1[ND:ND + NF * ED].T,
        W1[ND + NF * ED:].T,
        b1.reshape(H1, 1),
        W2.T,
        b2.reshape(H2, 1),
        Wf.T,
        bf.reshape(1, 1),
        lw_W.T,
        lw_b.reshape(1, 1),
    )
    return out_row.reshape(B, 1)


# speedup vs baseline: 14.4936x; 1.0265x over previous
import functools

import jax
import jax.numpy as jnp
from jax import lax
from jax.experimental import pallas as pl
from jax.experimental.pallas import tpu as pltpu
from jax.experimental.pallas import tpu_sc as plsc

B = 16384
ND = 13
NS = 26
V = 100001
VP = 100096
ED = 16
H1, H2 = 64, 32

NCORES = 2
NSUB = 16
NW = NCORES * NSUB

NG = 2
NF = NS // NG
BCH = 1024
NCHUNK = B // BCH
NTASK = NF * NCHUNK
TPW = -(-NTASK // NW)

ROWS_PER_FIELD = ED * (VP // 128)
SCRATCH_ROWS = NF * ROWS_PER_FIELD


def _detile_body(in_ref, out_ref):
    out_ref[...] = in_ref[0].reshape(ROWS_PER_FIELD, 128)


def _tc_detile(table_t, lo):
    return pl.pallas_call(
        _detile_body,
        grid=(NF,),
        in_specs=[pl.BlockSpec((1, ED, VP), lambda s: (s + lo, 0, 0))],
        out_specs=pl.BlockSpec((ROWS_PER_FIELD, 128), lambda s: (s, 0)),
        out_shape=jax.ShapeDtypeStruct((SCRATCH_ROWS, 128), jnp.float32),
    )(table_t)


def _sc_gather_build():
    mesh = plsc.VectorSubcoreMesh(core_axis_name="c", subcore_axis_name="s")

    @functools.partial(
        pl.kernel,
        mesh=mesh,
        compiler_params=pltpu.CompilerParams(use_tc_tiling_on_sc=False),
        out_type=(
            jax.ShapeDtypeStruct((NF, ED, NCHUNK, BCH), jnp.float32),
            jax.ShapeDtypeStruct((NF, NCHUNK, BCH), jnp.float32),
        ),
        scratch_types=[
            pltpu.VMEM((2, BCH), jnp.int32),
            pltpu.VMEM((2, BCH), jnp.int32),
            pltpu.VMEM((2, ED, BCH), jnp.float32),
            pltpu.VMEM((2, BCH), jnp.float32),
            pltpu.SemaphoreType.DMA,
            pltpu.SemaphoreType.DMA,
            pltpu.SemaphoreType.DMA,
            pltpu.SemaphoreType.DMA,
        ],
    )
    def sc_gather(table_hbm, linw_hbm, idx_hbm, idxw_hbm, embt_out, wv_out,
                  idx_v, idxw_v, rows_v, wv_v, sem_i, sem_g, sem_w, sem_wb):
        wid = lax.axis_index("s") * NCORES + lax.axis_index("c")
        for t in range(TPW):
            task = t * NW + wid
            b = t % 2

            @pl.when(task < NTASK)
            def _(t=t, task=task, b=b):
                s = task // NCHUNK
                c = task % NCHUNK
                boff = pl.multiple_of(c * BCH, BCH)
                pltpu.sync_copy(idx_hbm.at[s, pl.ds(boff, BCH)], idx_v.at[b])
                pltpu.sync_copy(idxw_hbm.at[s, pl.ds(boff, BCH)],
                                idxw_v.at[b])
                cps = [
                    pltpu.async_copy(
                        table_hbm.at[pl.ds((s * ED + e) * VP, VP)]
                        .at[idx_v.at[b]],
                        rows_v.at[b, e], sem_g)
                    for e in range(ED)
                ]
                cpw = pltpu.async_copy(linw_hbm.at[0].at[idxw_v.at[b]],
                                       wv_v.at[b], sem_w)
                for cp in cps:
                    cp.wait()
                cpw.wait()
                pltpu.sync_copy(rows_v.at[b], embt_out.at[s, :, c, :])
                pltpu.sync_copy(wv_v.at[b], wv_out.at[s, c, :])

    return sc_gather


_sc_gather = _sc_gather_build()

def _tc_mlp_body(xt_ref, emba_ref, embb_ref, wva_ref, wvb_ref,
                 w1dt_ref, w1at_ref, w1bt_ref, b1_ref,
                 w2t_ref, b2_ref, wft_ref, bf_ref, lwwt_ref, lwb_ref, out_ref):
    dense_t = xt_ref[:ND, :]
    emba = emba_ref[...].reshape(NF * ED, 128, 128).reshape(NF * ED, B)
    embb = embb_ref[...].reshape(NF * ED, 128, 128).reshape(NF * ED, B)
    wva = wva_ref[...].reshape(NF, 128, 128).reshape(NF, B)
    wvb = wvb_ref[...].reshape(NF, 128, 128).reshape(NF, B)
    wide = (
        jnp.dot(lwwt_ref[...], dense_t, preferred_element_type=jnp.float32)
        + lwb_ref[...]
        + jnp.sum(wva, axis=0, keepdims=True)
        + jnp.sum(wvb, axis=0, keepdims=True)
    )
    h = jnp.dot(w1dt_ref[...], dense_t, preferred_element_type=jnp.float32)
    h += jnp.dot(w1at_ref[...], emba, preferred_element_type=jnp.float32)
    h += jnp.dot(w1bt_ref[...], embb, preferred_element_type=jnp.float32)
    h = jax.nn.relu(h + b1_ref[...])
    h = jax.nn.relu(
        jnp.dot(w2t_ref[...], h, preferred_element_type=jnp.float32) + b2_ref[...]
    )
    deep = jnp.dot(wft_ref[...], h, preferred_element_type=jnp.float32) + bf_ref[...]
    out_ref[...] = jax.nn.sigmoid(0.5 * wide + 0.5 * deep)


def _tc_mlp(xt, emba, embb, wva, wvb, w1dt, w1at, w1bt, b1c, w2t, b2c,
            wft, bfc, lwwt, lwbc):
    return pl.pallas_call(
        _tc_mlp_body,
        out_shape=jax.ShapeDtypeStruct((1, B), jnp.float32),
    )(xt, emba.reshape(NF * ED * 128, 128), embb.reshape(NF * ED * 128, 128),
      wva.reshape(NF * 128, 128), wvb.reshape(NF * 128, 128),
      w1dt, w1at, w1bt, b1c, w2t, b2c, wft, bfc, lwwt, lwbc)


def kernel(inputs, embed_tables, linear_w, lw_W, lw_b, W1, b1, W2, b2, Wf, bf):
    table_t = jnp.transpose(embed_tables, (0, 2, 1))
    linw_t = jnp.transpose(linear_w, (1, 0))

    xt = jnp.transpose(inputs, (1, 0))
    idx_t = xt[ND:, :].astype(jnp.int32)
    idxw_t = idx_t + (jnp.arange(NS, dtype=jnp.int32) * V)[:, None]

    halves = []
    for g in range(NG):
        lo = g * NF
        table_f = _tc_detile(table_t, lo).reshape(-1)
        e4, w3 = _sc_gather(table_f, linw_t,
                            idx_t[lo:lo + NF], idxw_t[lo:lo + NF])
        halves.append((e4.reshape(NF * ED, B), w3.reshape(NF, B)))
    (emba, wva), (embb, wvb) = halves

    out_row = _tc_mlp(
        xt,
        emba,
        embb,
        wva,
        wvb,
        W1[:ND].T,
        W1[ND:ND + NF * ED].T,
        W1[ND + NF * ED:].T,
        b1.reshape(H1, 1),
        W2.T,
        b2.reshape(H2, 1),
        Wf.T,
        bf.reshape(1, 1),
        lw_W.T,
        lw_b.reshape(1, 1),
    )
    return out_row.reshape(B, 1)
